# Initial kernel scaffold; baseline (speedup 1.0000x reference)
import jax
import jax.numpy as jnp
from jax.experimental import pallas as pl


def kernel(x, edge_index, edge_type, W1, root1, b1, W2, root2, b2, Ws, bs):
    raise NotImplementedError("write your pallas kernel here")


# speedup vs baseline: 12.0648x; 12.0648x over previous
import functools

import jax
import jax.numpy as jnp
from jax import lax
from jax.experimental import pallas as pl
from jax.experimental.pallas import tpu as pltpu
from jax.experimental.pallas import tpu_sc as plsc

N = 10000
E = 320000
R = 8
NINE = 9

NC = 2
NS = 16
L = 16

E_PAD = 327680
EROWS = E_PAD // 128
CNT_TAB = 81920
TPB = CNT_TAB // NS
N_ACC = 10240

_mesh = functools.partial(
    plsc.VectorSubcoreMesh,
    core_axis_name="c", subcore_axis_name="s", num_cores=NC, num_subcores=NS)


def _sc_weights(widx2d):
    def body(widx_hbm, w_out, cnt_sh, invc_sh, widx_v, val_v, cb, sem):
        cid = lax.axis_index("c")
        sid = lax.axis_index("s")

        @pl.loop(0, TPB // L)
        def _z(i):
            cb[pl.ds(i * L, L)] = jnp.zeros((L,), jnp.float32)
        pltpu.sync_copy(cb, cnt_sh.at[pl.ds(sid * TPB, TPB)])

        @pl.loop(0, EROWS // NS)
        def _f(i):
            for j in range(128 // L):
                val_v[i, pl.ds(j * L, L)] = jnp.full((L,), 1.0, jnp.float32)
        plsc.subcore_barrier()

        pltpu.sync_copy(widx_hbm.at[pl.ds(sid * (EROWS // NS), EROWS // NS)],
                        widx_v)

        @pl.loop(0, EROWS // NS)
        def _cnt(i):
            pltpu.sync_copy(val_v.at[i], cnt_sh.at[widx_v.at[i]], add=True)
        plsc.subcore_barrier()

        pltpu.sync_copy(cnt_sh.at[pl.ds(sid * TPB, TPB)], cb)

        @pl.loop(0, TPB // L)
        def _r(i):
            sl = pl.ds(i * L, L)
            cb[sl] = 1.0 / jnp.maximum(cb[sl], 1.0)
        pltpu.sync_copy(cb, invc_sh.at[pl.ds(sid * TPB, TPB)])
        plsc.subcore_barrier()

        half = EROWS // (NC * NS)
        base = cid * (EROWS // NC) + sid * half
        pltpu.sync_copy(widx_hbm.at[pl.ds(base, half)],
                        widx_v.at[pl.ds(0, half)])

        @pl.loop(0, half)
        def _gw(i):
            pltpu.async_copy(invc_sh.at[widx_v.at[i]], val_v.at[i], sem).wait()
        pltpu.sync_copy(val_v.at[pl.ds(0, half)], w_out.at[pl.ds(base, half)])

    run = pl.kernel(
        body,
        out_type=jax.ShapeDtypeStruct((EROWS, 128), jnp.float32),
        mesh=_mesh(),
        scratch_types=[
            pltpu.VMEM_SHARED((CNT_TAB,), jnp.float32),
            pltpu.VMEM_SHARED((CNT_TAB,), jnp.float32),
            pltpu.VMEM((EROWS // NS, 128), jnp.int32),
            pltpu.VMEM((EROWS // NS, 128), jnp.float32),
            pltpu.VMEM((TPB,), jnp.float32),
            pltpu.SemaphoreType.DMA,
        ],
    )
    return run(widx2d)


def _sc_agg(rowidx2d, dst2d, w2d, hw_flat):
    H = 128
    half = EROWS // (NC * NS)

    def body(ridx_hbm, didx_hbm, w_hbm, hw_hbm, out_hbm,
             acc_sh, ridx_v, didx_v, w_v, rows_v, sem):
        cid = lax.axis_index("c")
        sid = lax.axis_index("s")

        @pl.loop(0, 128)
        def _z(i):
            for j in range(H // L):
                rows_v[i, pl.ds(j * L, L)] = jnp.zeros((L,), jnp.float32)
        for k in range(N_ACC // NS // 128):
            pltpu.sync_copy(rows_v,
                            acc_sh.at[pl.ds(sid * (N_ACC // NS) + k * 128, 128)])
        plsc.subcore_barrier()

        base = cid * (EROWS // NC) + sid * half
        pltpu.sync_copy(ridx_hbm.at[pl.ds(base, half)], ridx_v)
        pltpu.sync_copy(didx_hbm.at[pl.ds(base, half)], didx_v)
        pltpu.sync_copy(w_hbm.at[pl.ds(base, half)], w_v)

        @pl.loop(0, half)
        def _chunk(ch):
            pltpu.async_copy(hw_hbm.at[ridx_v.at[ch]], rows_v, sem).wait()

            @pl.loop(0, 128 // L)
            def _grp(g):
                wv = w_v[ch, pl.ds(g * L, L)]
                for e in range(L):
                    w = wv[e]
                    for j in range(H // L):
                        sl = pl.ds(j * L, L)
                        rows_v[g * L + e, sl] = rows_v[g * L + e, sl] * w
            pltpu.sync_copy(rows_v, acc_sh.at[didx_v.at[ch]], add=True)
        plsc.subcore_barrier()

        for k in range(N_ACC // NS // 128):
            off = sid * (N_ACC // NS) + k * 128
            pltpu.sync_copy(acc_sh.at[pl.ds(off, 128)],
                            out_hbm.at[cid, pl.ds(off, 128)])

    run = pl.kernel(
        body,
        out_type=jax.ShapeDtypeStruct((NC, N_ACC, 128), jnp.float32),
        mesh=_mesh(),
        scratch_types=[
            pltpu.VMEM_SHARED((N_ACC, H), jnp.float32),
            pltpu.VMEM((half, 128), jnp.int32),
            pltpu.VMEM((half, 128), jnp.int32),
            pltpu.VMEM((half, 128), jnp.float32),
            pltpu.VMEM((128, H), jnp.float32),
            pltpu.SemaphoreType.DMA,
        ],
    )
    return run(rowidx2d, dst2d, w2d, hw_flat)


def _tc_mm1(x, wcat):
    def body(x_ref, w_ref, o_ref):
        o_ref[...] = jnp.dot(x_ref[...], w_ref[...],
                             preferred_element_type=jnp.float32)
    return pl.pallas_call(
        body,
        grid=(50,),
        in_specs=[
            pl.BlockSpec((200, 128), lambda i: (i, 0)),
            pl.BlockSpec((128, NINE * 128), lambda i: (0, 0)),
        ],
        out_specs=pl.BlockSpec((200, NINE * 128), lambda i: (i, 0)),
        out_shape=jax.ShapeDtypeStruct((N, NINE * 128), jnp.float32),
    )(x, wcat)


def _tc_fuse2(hw1, parts, b1, wcat2):
    def body(hw_ref, p_ref, b_ref, w_ref, o_ref):
        h = jax.nn.relu(hw_ref[:, :128] + b_ref[0] + p_ref[0] + p_ref[1])
        o_ref[...] = jnp.dot(h, w_ref[...], preferred_element_type=jnp.float32)
    return pl.pallas_call(
        body,
        grid=(50,),
        in_specs=[
            pl.BlockSpec((200, NINE * 128), lambda i: (i, 0)),
            pl.BlockSpec((NC, 200, 128), lambda i: (0, i, 0)),
            pl.BlockSpec((1, 128), lambda i: (0, 0)),
            pl.BlockSpec((128, NINE * 128), lambda i: (0, 0)),
        ],
        out_specs=pl.BlockSpec((200, NINE * 128), lambda i: (i, 0)),
        out_shape=jax.ShapeDtypeStruct((N, NINE * 128), jnp.float32),
    )(hw1, parts, b1, wcat2)


def _tc_fuse3(hw2, parts, b2, ws, bs):
    def body(hw_ref, p_ref, b_ref, w_ref, bs_ref, o_ref):
        h = jax.nn.relu(hw_ref[:, :64] + b_ref[0]
                        + p_ref[0, :, :64] + p_ref[1, :, :64])
        raw = jnp.dot(h, w_ref[...], preferred_element_type=jnp.float32)
        o_ref[...] = jax.nn.sigmoid(raw + bs_ref[0])
    return pl.pallas_call(
        body,
        grid=(50,),
        in_specs=[
            pl.BlockSpec((200, NINE * 128), lambda i: (i, 0)),
            pl.BlockSpec((NC, 200, 128), lambda i: (0, i, 0)),
            pl.BlockSpec((1, 64), lambda i: (0, 0)),
            pl.BlockSpec((64, 1), lambda i: (0, 0)),
            pl.BlockSpec((1, 1), lambda i: (0, 0)),
        ],
        out_specs=pl.BlockSpec((200, 1), lambda i: (i, 0)),
        out_shape=jax.ShapeDtypeStruct((N, 1), jnp.float32),
    )(hw2, parts, b2, ws, bs)


def kernel(x, edge_index, edge_type, W1, root1, b1, W2, root2, b2, Ws, bs):
    src = edge_index[0].astype(jnp.int32)
    dst = edge_index[1].astype(jnp.int32)
    typ = edge_type.astype(jnp.int32)

    pad = E_PAD - E
    widx = jnp.concatenate(
        [typ * N + dst, jnp.full((pad,), CNT_TAB - 1, jnp.int32)])
    rowidx = jnp.concatenate(
        [src * NINE + typ + 1, jnp.zeros((pad,), jnp.int32)])
    dstp = jnp.concatenate(
        [dst, jnp.full((pad,), N_ACC - 1, jnp.int32)])
    widx2d = widx.reshape(EROWS, 128)
    rowidx2d = rowidx.reshape(EROWS, 128)
    dst2d = dstp.reshape(EROWS, 128)

    wcat1 = jnp.concatenate(
        [root1[:, None, :], jnp.moveaxis(W1, 0, 1)], axis=1).reshape(128, NINE * 128)
    wcat2 = jnp.pad(
        jnp.concatenate([root2[:, None, :], jnp.moveaxis(W2, 0, 1)], axis=1),
        ((0, 0), (0, 0), (0, 64))).reshape(128, NINE * 128)

    hw1 = _tc_mm1(x, wcat1)
    w2d = _sc_weights(widx2d)
    parts1 = _sc_agg(rowidx2d, dst2d, w2d,
                     hw1.reshape(N * NINE, 128))
    hw2 = _tc_fuse2(hw1, parts1, b1.reshape(1, 128), wcat2)
    parts2 = _sc_agg(rowidx2d, dst2d, w2d,
                     hw2.reshape(N * NINE, 128))
    out = _tc_fuse3(hw2, parts2, b2.reshape(1, 64),
                    Ws, bs.reshape(1, 1))
    return jnp.squeeze(out, axis=-1)


# speedup vs baseline: 14.0102x; 1.1612x over previous
import functools

import jax
import jax.numpy as jnp
from jax import lax
from jax.experimental import pallas as pl
from jax.experimental.pallas import tpu as pltpu
from jax.experimental.pallas import tpu_sc as plsc

N = 10000
E = 320000
R = 8
NINE = 9

NC = 2
NS = 16
L = 16

E_PAD = 327680
EROWS = E_PAD // 128
CNT_TAB = 81920
TPB = CNT_TAB // NS
N_ACC = 10240

_mesh = functools.partial(
    plsc.VectorSubcoreMesh,
    core_axis_name="c", subcore_axis_name="s", num_cores=NC, num_subcores=NS)


def _sc_weights(widx2d):
    def body(widx_hbm, w_out, cnt_sh, invc_sh, widx_v, val_v, cb, sem):
        cid = lax.axis_index("c")
        sid = lax.axis_index("s")

        @pl.loop(0, TPB // L)
        def _z(i):
            cb[pl.ds(i * L, L)] = jnp.zeros((L,), jnp.float32)
        pltpu.sync_copy(cb, cnt_sh.at[pl.ds(sid * TPB, TPB)])

        @pl.loop(0, EROWS // NS)
        def _f(i):
            for j in range(128 // L):
                val_v[i, pl.ds(j * L, L)] = jnp.full((L,), 1.0, jnp.float32)
        plsc.subcore_barrier()

        pltpu.sync_copy(widx_hbm.at[pl.ds(sid * (EROWS // NS), EROWS // NS)],
                        widx_v)

        @pl.loop(0, EROWS // NS)
        def _cnt(i):
            pltpu.sync_copy(val_v.at[i], cnt_sh.at[widx_v.at[i]], add=True)
        plsc.subcore_barrier()

        pltpu.sync_copy(cnt_sh.at[pl.ds(sid * TPB, TPB)], cb)

        @pl.loop(0, TPB // L)
        def _r(i):
            sl = pl.ds(i * L, L)
            cb[sl] = 1.0 / jnp.maximum(cb[sl], 1.0)
        pltpu.sync_copy(cb, invc_sh.at[pl.ds(sid * TPB, TPB)])
        plsc.subcore_barrier()

        half = EROWS // (NC * NS)
        base = cid * (EROWS // NC) + sid * half
        pltpu.sync_copy(widx_hbm.at[pl.ds(base, half)],
                        widx_v.at[pl.ds(0, half)])

        @pl.loop(0, half)
        def _gw(i):
            pltpu.async_copy(invc_sh.at[widx_v.at[i]], val_v.at[i], sem).wait()
        pltpu.sync_copy(val_v.at[pl.ds(0, half)], w_out.at[pl.ds(base, half)])

    run = pl.kernel(
        body,
        out_type=jax.ShapeDtypeStruct((EROWS, 128), jnp.float32),
        mesh=_mesh(),
        scratch_types=[
            pltpu.VMEM_SHARED((CNT_TAB,), jnp.float32),
            pltpu.VMEM_SHARED((CNT_TAB,), jnp.float32),
            pltpu.VMEM((EROWS // NS, 128), jnp.int32),
            pltpu.VMEM((EROWS // NS, 128), jnp.float32),
            pltpu.VMEM((TPB,), jnp.float32),
            pltpu.SemaphoreType.DMA,
        ],
    )
    return run(widx2d)


def _sc_agg(idxw, w2d, hw_flat):
    H = 128
    half = EROWS // (NC * NS)
    NG = half // 8

    def body(idxw_hbm, w_hbm, hw_hbm, out_hbm,
             acc_sh, idxA, idxB, wA, wB, rows0, rows1,
             gsem0, gsem1, ssem0, ssem1, isemA, isemB):
        cid = lax.axis_index("c")
        sid = lax.axis_index("s")
        rows = (rows0, rows1)
        gsem = (gsem0, gsem1)
        ssem = (ssem0, ssem1)

        @pl.loop(0, 128)
        def _z(i):
            for j in range(H // L):
                rows0[i, pl.ds(j * L, L)] = jnp.zeros((L,), jnp.float32)
        for k in range(N_ACC // NS // 128):
            pltpu.sync_copy(rows0,
                            acc_sh.at[pl.ds(sid * (N_ACC // NS) + k * 128, 128)])
        plsc.subcore_barrier()

        base = cid * (EROWS // NC) + sid * half

        def i_start(blk, wblk, sem, ch):
            pltpu.async_copy(idxw_hbm.at[pl.ds(base + ch, 4)], blk, sem)
            pltpu.async_copy(w_hbm.at[pl.ds(base + ch, 4)], wblk, sem)

        def i_wait(blk, wblk, sem, ch):
            pltpu.make_async_copy(idxw_hbm.at[pl.ds(base + ch, 4)], blk,
                                  sem).wait()
            pltpu.make_async_copy(w_hbm.at[pl.ds(base + ch, 4)], wblk,
                                  sem).wait()

        def g_start(blk, r, b):
            pltpu.async_copy(hw_hbm.at[blk.at[r, 0]], rows[b], gsem[b])

        def g_wait(blk, r, b):
            pltpu.make_async_copy(hw_hbm.at[blk.at[r, 0]], rows[b],
                                  gsem[b]).wait()

        def s_start(blk, r, b):
            pltpu.async_copy(rows[b], acc_sh.at[blk.at[r, 1]], ssem[b],
                             add=True)

        def s_wait(blk, r, b):
            pltpu.make_async_copy(rows[b], acc_sh.at[blk.at[r, 1]],
                                  ssem[b]).wait()

        def compute(blk, wblk, r, b):
            @pl.loop(0, 128 // L)
            def _grp(g):
                wv = wblk[r, pl.ds(g * L, L)]
                for e in range(L):
                    w = wv[e]
                    for j in range(H // L):
                        sl = pl.ds(j * L, L)
                        rows[b][g * L + e, sl] = rows[b][g * L + e, sl] * w

        i_start(idxA, wA, isemA, 0)
        i_start(idxB, wB, isemB, 4)
        i_wait(idxA, wA, isemA, 0)
        g_start(idxA, 0, 0)

        @pl.loop(0, NG)
        def _grp8(p):
            ch0 = p * 8
            for k in range(8):
                b = k % 2
                blk, wblk = (idxA, wA) if k < 4 else (idxB, wB)
                r = k % 4
                g_wait(blk, r, b)
                if k > 0:
                    blkp = idxA if (k - 1) < 4 else idxB
                    s_wait(blkp, (k - 1) % 4, 1 - b)
                else:
                    @pl.when(p > 0)
                    def _():
                        s_wait(idxB, 3, 1)
                        i_start(idxB, wB, isemB, ch0 + 4)
                if k == 4:
                    @pl.when(p < NG - 1)
                    def _():
                        i_start(idxA, wA, isemA, ch0 + 8)
                if k == 3:
                    i_wait(idxB, wB, isemB, ch0 + 4)
                if k < 7:
                    nblk = idxA if (k + 1) < 4 else idxB
                    g_start(nblk, (k + 1) % 4, 1 - b)
                else:
                    @pl.when(p < NG - 1)
                    def _():
                        i_wait(idxA, wA, isemA, ch0 + 8)
                        g_start(idxA, 0, 1 - b)
                compute(blk, wblk, r, b)
                s_start(blk, r, b)

        s_wait(idxB, 3, 1)
        plsc.subcore_barrier()

        for k in range(N_ACC // NS // 128):
            off = sid * (N_ACC // NS) + k * 128
            pltpu.sync_copy(acc_sh.at[pl.ds(off, 128)],
                            out_hbm.at[cid, pl.ds(off, 128)])

    run = pl.kernel(
        body,
        out_type=jax.ShapeDtypeStruct((NC, N_ACC, 128), jnp.float32),
        mesh=_mesh(),
        scratch_types=[
            pltpu.VMEM_SHARED((N_ACC, H), jnp.float32),
            pltpu.VMEM((4, 2, 128), jnp.int32),
            pltpu.VMEM((4, 2, 128), jnp.int32),
            pltpu.VMEM((4, 128), jnp.float32),
            pltpu.VMEM((4, 128), jnp.float32),
            pltpu.VMEM((128, H), jnp.float32),
            pltpu.VMEM((128, H), jnp.float32),
            pltpu.SemaphoreType.DMA,
            pltpu.SemaphoreType.DMA,
            pltpu.SemaphoreType.DMA,
            pltpu.SemaphoreType.DMA,
            pltpu.SemaphoreType.DMA,
            pltpu.SemaphoreType.DMA,
        ],
    )
    return run(idxw, w2d, hw_flat)


def _tc_mm1(x, wcat):
    def body(x_ref, w_ref, o_ref):
        o_ref[...] = jnp.dot(x_ref[...], w_ref[...],
                             preferred_element_type=jnp.float32)
    return pl.pallas_call(
        body,
        grid=(50,),
        in_specs=[
            pl.BlockSpec((200, 128), lambda i: (i, 0)),
            pl.BlockSpec((128, NINE * 128), lambda i: (0, 0)),
        ],
        out_specs=pl.BlockSpec((200, NINE * 128), lambda i: (i, 0)),
        out_shape=jax.ShapeDtypeStruct((N, NINE * 128), jnp.float32),
    )(x, wcat)


def _tc_fuse2(hw1, parts, b1, wcat2):
    def body(hw_ref, p_ref, b_ref, w_ref, o_ref):
        h = jax.nn.relu(hw_ref[:, :128] + b_ref[0] + p_ref[0] + p_ref[1])
        o_ref[...] = jnp.dot(h, w_ref[...], preferred_element_type=jnp.float32)
    return pl.pallas_call(
        body,
        grid=(50,),
        in_specs=[
            pl.BlockSpec((200, NINE * 128), lambda i: (i, 0)),
            pl.BlockSpec((NC, 200, 128), lambda i: (0, i, 0)),
            pl.BlockSpec((1, 128), lambda i: (0, 0)),
            pl.BlockSpec((128, NINE * 128), lambda i: (0, 0)),
        ],
        out_specs=pl.BlockSpec((200, NINE * 128), lambda i: (i, 0)),
        out_shape=jax.ShapeDtypeStruct((N, NINE * 128), jnp.float32),
    )(hw1, parts, b1, wcat2)


def _tc_fuse3(hw2, parts, b2, ws, bs):
    def body(hw_ref, p_ref, b_ref, w_ref, bs_ref, o_ref):
        h = jax.nn.relu(hw_ref[:, :64] + b_ref[0]
                        + p_ref[0, :, :64] + p_ref[1, :, :64])
        raw = jnp.dot(h, w_ref[...], preferred_element_type=jnp.float32)
        o_ref[...] = jax.nn.sigmoid(raw + bs_ref[0])
    return pl.pallas_call(
        body,
        grid=(50,),
        in_specs=[
            pl.BlockSpec((200, NINE * 128), lambda i: (i, 0)),
            pl.BlockSpec((NC, 200, 128), lambda i: (0, i, 0)),
            pl.BlockSpec((1, 64), lambda i: (0, 0)),
            pl.BlockSpec((64, 1), lambda i: (0, 0)),
            pl.BlockSpec((1, 1), lambda i: (0, 0)),
        ],
        out_specs=pl.BlockSpec((200, 1), lambda i: (i, 0)),
        out_shape=jax.ShapeDtypeStruct((N, 1), jnp.float32),
    )(hw2, parts, b2, ws, bs)


def kernel(x, edge_index, edge_type, W1, root1, b1, W2, root2, b2, Ws, bs):
    src = edge_index[0].astype(jnp.int32)
    dst = edge_index[1].astype(jnp.int32)
    typ = edge_type.astype(jnp.int32)

    pad = E_PAD - E
    widx = jnp.concatenate(
        [typ * N + dst, jnp.full((pad,), CNT_TAB - 1, jnp.int32)])
    rowidx = jnp.concatenate(
        [src * NINE + typ + 1, jnp.zeros((pad,), jnp.int32)])
    dstp = jnp.concatenate(
        [dst, jnp.full((pad,), N_ACC - 1, jnp.int32)])
    widx2d = widx.reshape(EROWS, 128)
    rowidx2d = rowidx.reshape(EROWS, 128)
    dst2d = dstp.reshape(EROWS, 128)

    wcat1 = jnp.concatenate(
        [root1[:, None, :], jnp.moveaxis(W1, 0, 1)], axis=1).reshape(128, NINE * 128)
    wcat2 = jnp.pad(
        jnp.concatenate([root2[:, None, :], jnp.moveaxis(W2, 0, 1)], axis=1),
        ((0, 0), (0, 0), (0, 64))).reshape(128, NINE * 128)

    hw1 = _tc_mm1(x, wcat1)
    w2d = _sc_weights(widx2d)
    idxw = jnp.stack([rowidx2d, dst2d], axis=1)
    parts1 = _sc_agg(idxw, w2d, hw1.reshape(N * NINE, 128))
    hw2 = _tc_fuse2(hw1, parts1, b1.reshape(1, 128), wcat2)
    parts2 = _sc_agg(idxw, w2d, hw2.reshape(N * NINE, 128))
    out = _tc_fuse3(hw2, parts2, b2.reshape(1, 64),
                    Ws, bs.reshape(1, 1))
    return jnp.squeeze(out, axis=-1)


# speedup vs baseline: 29.6854x; 2.1188x over previous
import functools

import jax
import jax.numpy as jnp
from jax import lax
from jax.experimental import pallas as pl
from jax.experimental.pallas import tpu as pltpu
from jax.experimental.pallas import tpu_sc as plsc

N = 10000
E = 320000
R = 8
NINE = 9

NC = 2
NS = 16
L = 16

E_PAD = 327680
EROWS = E_PAD // 128
CNT_TAB = 81920
TPB = CNT_TAB // NS
N_ACC = 10240

_mesh = functools.partial(
    plsc.VectorSubcoreMesh,
    core_axis_name="c", subcore_axis_name="s", num_cores=NC, num_subcores=NS)


def _sc_weights(widx2d):
    def body(widx_hbm, w_out, cnt_sh, invc_sh, widx_v, val_v, cb, sem):
        cid = lax.axis_index("c")
        sid = lax.axis_index("s")

        @pl.loop(0, TPB // L)
        def _z(i):
            cb[pl.ds(i * L, L)] = jnp.zeros((L,), jnp.float32)
        pltpu.sync_copy(cb, cnt_sh.at[pl.ds(sid * TPB, TPB)])

        @pl.loop(0, EROWS // NS)
        def _f(i):
            for j in range(128 // L):
                val_v[i, pl.ds(j * L, L)] = jnp.full((L,), 1.0, jnp.float32)
        plsc.subcore_barrier()

        pltpu.sync_copy(widx_hbm.at[pl.ds(sid * (EROWS // NS), EROWS // NS)],
                        widx_v)

        @pl.loop(0, EROWS // NS)
        def _cnt(i):
            pltpu.sync_copy(val_v.at[i], cnt_sh.at[widx_v.at[i]], add=True)
        plsc.subcore_barrier()

        pltpu.sync_copy(cnt_sh.at[pl.ds(sid * TPB, TPB)], cb)

        @pl.loop(0, TPB // L)
        def _r(i):
            sl = pl.ds(i * L, L)
            cb[sl] = 1.0 / jnp.maximum(cb[sl], 1.0)
        pltpu.sync_copy(cb, invc_sh.at[pl.ds(sid * TPB, TPB)])
        plsc.subcore_barrier()

        half = EROWS // (NC * NS)
        base = cid * (EROWS // NC) + sid * half
        pltpu.sync_copy(widx_hbm.at[pl.ds(base, half)],
                        widx_v.at[pl.ds(0, half)])

        @pl.loop(0, half)
        def _gw(i):
            pltpu.async_copy(invc_sh.at[widx_v.at[i]], val_v.at[i], sem).wait()
        pltpu.sync_copy(val_v.at[pl.ds(0, half)], w_out.at[pl.ds(base, half)])

    run = pl.kernel(
        body,
        out_type=jax.ShapeDtypeStruct((EROWS, 128), jnp.float32),
        mesh=_mesh(),
        scratch_types=[
            pltpu.VMEM_SHARED((CNT_TAB,), jnp.float32),
            pltpu.VMEM_SHARED((CNT_TAB,), jnp.float32),
            pltpu.VMEM((EROWS // NS, 128), jnp.int32),
            pltpu.VMEM((EROWS // NS, 128), jnp.float32),
            pltpu.VMEM((TPB,), jnp.float32),
            pltpu.SemaphoreType.DMA,
        ],
    )
    return run(widx2d)


def _sc_agg(idxw, w2d, hw_flat):
    H = 128
    half = EROWS // (NC * NS)
    NG = half // 8

    def body(idxw_hbm, w_hbm, hw_hbm, out_hbm,
             acc_sh, idxA, idxB, wA, wB, rows0, rows1,
             gsem0, gsem1, ssem0, ssem1, isemA, isemB):
        cid = lax.axis_index("c")
        sid = lax.axis_index("s")
        rows = (rows0, rows1)
        gsem = (gsem0, gsem1)
        ssem = (ssem0, ssem1)

        @pl.loop(0, 128)
        def _z(i):
            for j in range(H // L):
                rows0[i, pl.ds(j * L, L)] = jnp.zeros((L,), jnp.float32)
        for k in range(N_ACC // NS // 128):
            pltpu.sync_copy(rows0,
                            acc_sh.at[pl.ds(sid * (N_ACC // NS) + k * 128, 128)])
        plsc.subcore_barrier()

        base = cid * (EROWS // NC) + sid * half

        def i_start(blk, wblk, sem, ch):
            pltpu.async_copy(idxw_hbm.at[pl.ds(base + ch, 4)], blk, sem)
            pltpu.async_copy(w_hbm.at[pl.ds(base + ch, 4)], wblk, sem)

        def i_wait(blk, wblk, sem, ch):
            pltpu.make_async_copy(idxw_hbm.at[pl.ds(base + ch, 4)], blk,
                                  sem).wait()
            pltpu.make_async_copy(w_hbm.at[pl.ds(base + ch, 4)], wblk,
                                  sem).wait()

        def g_start(blk, r, b):
            pltpu.async_copy(hw_hbm.at[blk.at[r, 0]], rows[b], gsem[b])

        def g_wait(blk, r, b):
            pltpu.make_async_copy(hw_hbm.at[blk.at[r, 0]], rows[b],
                                  gsem[b]).wait()

        def s_start(blk, r, b):
            pltpu.async_copy(rows[b], acc_sh.at[blk.at[r, 1]], ssem[b],
                             add=True)

        def s_wait(blk, r, b):
            pltpu.make_async_copy(rows[b], acc_sh.at[blk.at[r, 1]],
                                  ssem[b]).wait()

        def compute(blk, wblk, r, b):
            @pl.loop(0, 128 // L)
            def _grp(g):
                wv = wblk[r, pl.ds(g * L, L)]
                for e in range(L):
                    w = wv[e]
                    for j in range(H // L):
                        sl = pl.ds(j * L, L)
                        rows[b][g * L + e, sl] = rows[b][g * L + e, sl] * w

        i_start(idxA, wA, isemA, 0)
        i_start(idxB, wB, isemB, 4)
        i_wait(idxA, wA, isemA, 0)
        g_start(idxA, 0, 0)

        @pl.loop(0, NG)
        def _grp8(p):
            ch0 = p * 8
            for k in range(8):
                b = k % 2
                blk, wblk = (idxA, wA) if k < 4 else (idxB, wB)
                r = k % 4
                g_wait(blk, r, b)
                if k > 0:
                    blkp = idxA if (k - 1) < 4 else idxB
                    s_wait(blkp, (k - 1) % 4, 1 - b)
                else:
                    @pl.when(p > 0)
                    def _():
                        s_wait(idxB, 3, 1)
                        i_start(idxB, wB, isemB, ch0 + 4)
                if k == 4:
                    @pl.when(p < NG - 1)
                    def _():
                        i_start(idxA, wA, isemA, ch0 + 8)
                if k == 3:
                    i_wait(idxB, wB, isemB, ch0 + 4)
                if k < 7:
                    nblk = idxA if (k + 1) < 4 else idxB
                    g_start(nblk, (k + 1) % 4, 1 - b)
                else:
                    @pl.when(p < NG - 1)
                    def _():
                        i_wait(idxA, wA, isemA, ch0 + 8)
                        g_start(idxA, 0, 1 - b)
                compute(blk, wblk, r, b)
                s_start(blk, r, b)

        s_wait(idxB, 3, 1)
        plsc.subcore_barrier()

        for k in range(N_ACC // NS // 128):
            off = sid * (N_ACC // NS) + k * 128
            pltpu.sync_copy(acc_sh.at[pl.ds(off, 128)],
                            out_hbm.at[cid, pl.ds(off, 128)])

    run = pl.kernel(
        body,
        out_type=jax.ShapeDtypeStruct((NC, N_ACC, 128), jnp.float32),
        mesh=_mesh(),
        scratch_types=[
            pltpu.VMEM_SHARED((N_ACC, H), jnp.float32),
            pltpu.VMEM((4, 2, 128), jnp.int32),
            pltpu.VMEM((4, 2, 128), jnp.int32),
            pltpu.VMEM((4, 128), jnp.float32),
            pltpu.VMEM((4, 128), jnp.float32),
            pltpu.VMEM((128, H), jnp.float32),
            pltpu.VMEM((128, H), jnp.float32),
            pltpu.SemaphoreType.DMA,
            pltpu.SemaphoreType.DMA,
            pltpu.SemaphoreType.DMA,
            pltpu.SemaphoreType.DMA,
            pltpu.SemaphoreType.DMA,
            pltpu.SemaphoreType.DMA,
        ],
    )
    return run(idxw, w2d, hw_flat)


def _tc_mm1(x, wcat):
    def body(x_ref, w_ref, o_ref):
        o_ref[...] = jnp.dot(x_ref[...], w_ref[...],
                             preferred_element_type=jnp.float32)
    return pl.pallas_call(
        body,
        grid=(50,),
        in_specs=[
            pl.BlockSpec((200, 128), lambda i: (i, 0)),
            pl.BlockSpec((128, NINE * 128), lambda i: (0, 0)),
        ],
        out_specs=pl.BlockSpec((200, NINE * 128), lambda i: (i, 0)),
        out_shape=jax.ShapeDtypeStruct((N, NINE * 128), jnp.float32),
    )(x, wcat)


def _tc_fuse2(hw1, parts, b1, wcat2):
    def body(hw_ref, p_ref, b_ref, w_ref, o_ref):
        h = jax.nn.relu(hw_ref[:, :128] + b_ref[0] + p_ref[0] + p_ref[1])
        o_ref[...] = jnp.dot(h, w_ref[...], preferred_element_type=jnp.float32)
    return pl.pallas_call(
        body,
        grid=(50,),
        in_specs=[
            pl.BlockSpec((200, NINE * 128), lambda i: (i, 0)),
            pl.BlockSpec((NC, 200, 128), lambda i: (0, i, 0)),
            pl.BlockSpec((1, 128), lambda i: (0, 0)),
            pl.BlockSpec((128, NINE * 128), lambda i: (0, 0)),
        ],
        out_specs=pl.BlockSpec((200, NINE * 128), lambda i: (i, 0)),
        out_shape=jax.ShapeDtypeStruct((N, NINE * 128), jnp.float32),
    )(hw1, parts, b1, wcat2)


def _tc_fuse3(hw2, parts, b2, ws, bs):
    def body(hw_ref, p_ref, b_ref, w_ref, bs_ref, o_ref):
        h = jax.nn.relu(hw_ref[:, :64] + b_ref[0]
                        + p_ref[0, :, :64] + p_ref[1, :, :64])
        raw = jnp.dot(h, w_ref[...], preferred_element_type=jnp.float32)
        o_ref[...] = jax.nn.sigmoid(raw + bs_ref[0])
    return pl.pallas_call(
        body,
        grid=(50,),
        in_specs=[
            pl.BlockSpec((200, NINE * 128), lambda i: (i, 0)),
            pl.BlockSpec((NC, 200, 128), lambda i: (0, i, 0)),
            pl.BlockSpec((1, 64), lambda i: (0, 0)),
            pl.BlockSpec((64, 1), lambda i: (0, 0)),
            pl.BlockSpec((1, 1), lambda i: (0, 0)),
        ],
        out_specs=pl.BlockSpec((200, 1), lambda i: (i, 0)),
        out_shape=jax.ShapeDtypeStruct((N, 1), jnp.float32),
    )(hw2, parts, b2, ws, bs)


def kernel(x, edge_index, edge_type, W1, root1, b1, W2, root2, b2, Ws, bs):
    src = edge_index[0].astype(jnp.int32)
    dst = edge_index[1].astype(jnp.int32)
    typ = edge_type.astype(jnp.int32)

    pad = E_PAD - E
    ar = jnp.arange(pad, dtype=jnp.int32)
    widx = jnp.concatenate([typ * N + dst, 80000 + ar % (CNT_TAB - 80000)])
    rowidx = jnp.concatenate([src * NINE + typ + 1, ar % (N * NINE)])
    dstp = jnp.concatenate([dst, N + ar % (N_ACC - N)])
    widx2d = widx.reshape(EROWS, 128)
    rowidx2d = rowidx.reshape(EROWS, 128)
    dst2d = dstp.reshape(EROWS, 128)

    wcat1 = jnp.concatenate(
        [root1[:, None, :], jnp.moveaxis(W1, 0, 1)], axis=1).reshape(128, NINE * 128)
    wcat2 = jnp.pad(
        jnp.concatenate([root2[:, None, :], jnp.moveaxis(W2, 0, 1)], axis=1),
        ((0, 0), (0, 0), (0, 64))).reshape(128, NINE * 128)

    hw1 = _tc_mm1(x, wcat1)
    w2d = _sc_weights(widx2d)
    idxw = jnp.stack([rowidx2d, dst2d], axis=1)
    parts1 = _sc_agg(idxw, w2d, hw1.reshape(N * NINE, 128))
    hw2 = _tc_fuse2(hw1, parts1, b1.reshape(1, 128), wcat2)
    parts2 = _sc_agg(idxw, w2d, hw2.reshape(N * NINE, 128))
    out = _tc_fuse3(hw2, parts2, b2.reshape(1, 64),
                    Ws, bs.reshape(1, 1))
    return jnp.squeeze(out, axis=-1)


# speedup vs baseline: 30.7643x; 1.0363x over previous
import functools

import jax
import jax.numpy as jnp
from jax import lax
from jax.experimental import pallas as pl
from jax.experimental.pallas import tpu as pltpu
from jax.experimental.pallas import tpu_sc as plsc

N = 10000
E = 320000
R = 8
NINE = 9

NC = 2
NS = 16
L = 16

E_PAD = 327680
EROWS = E_PAD // 128
CNT_TAB = 81920
TPB = CNT_TAB // NS
N_ACC = 10240

_mesh = functools.partial(
    plsc.VectorSubcoreMesh,
    core_axis_name="c", subcore_axis_name="s", num_cores=NC, num_subcores=NS)


def _sc_weights(widx2d):
    def body(widx_hbm, w_out, cnt_sh, invc_sh, widx_v, val_v, cb, sem):
        cid = lax.axis_index("c")
        sid = lax.axis_index("s")

        @pl.loop(0, TPB // L)
        def _z(i):
            cb[pl.ds(i * L, L)] = jnp.zeros((L,), jnp.float32)
        pltpu.sync_copy(cb, cnt_sh.at[pl.ds(sid * TPB, TPB)])

        @pl.loop(0, EROWS // NS)
        def _f(i):
            for j in range(128 // L):
                val_v[i, pl.ds(j * L, L)] = jnp.full((L,), 1.0, jnp.float32)
        plsc.subcore_barrier()

        pltpu.sync_copy(widx_hbm.at[pl.ds(sid * (EROWS // NS), EROWS // NS)],
                        widx_v)

        @pl.loop(0, EROWS // NS)
        def _cnt(i):
            pltpu.async_copy(val_v.at[i], cnt_sh.at[widx_v.at[i]], sem,
                             add=True)

        @pl.loop(0, EROWS // NS)
        def _cntw(i):
            pltpu.make_async_copy(val_v.at[i], cnt_sh.at[widx_v.at[i]],
                                  sem).wait()
        plsc.subcore_barrier()

        pltpu.sync_copy(cnt_sh.at[pl.ds(sid * TPB, TPB)], cb)

        @pl.loop(0, TPB // L)
        def _r(i):
            sl = pl.ds(i * L, L)
            cb[sl] = 1.0 / jnp.maximum(cb[sl], 1.0)
        pltpu.sync_copy(cb, invc_sh.at[pl.ds(sid * TPB, TPB)])
        plsc.subcore_barrier()

        half = EROWS // (NC * NS)
        base = cid * (EROWS // NC) + sid * half
        pltpu.sync_copy(widx_hbm.at[pl.ds(base, half)],
                        widx_v.at[pl.ds(0, half)])

        @pl.loop(0, half)
        def _gw(i):
            pltpu.async_copy(invc_sh.at[widx_v.at[i]], val_v.at[i], sem)

        @pl.loop(0, half)
        def _gww(i):
            pltpu.make_async_copy(invc_sh.at[widx_v.at[i]], val_v.at[i],
                                  sem).wait()
        pltpu.sync_copy(val_v.at[pl.ds(0, half)], w_out.at[pl.ds(base, half)])

    run = pl.kernel(
        body,
        out_type=jax.ShapeDtypeStruct((EROWS, 128), jnp.float32),
        mesh=_mesh(),
        scratch_types=[
            pltpu.VMEM_SHARED((CNT_TAB,), jnp.float32),
            pltpu.VMEM_SHARED((CNT_TAB,), jnp.float32),
            pltpu.VMEM((EROWS // NS, 128), jnp.int32),
            pltpu.VMEM((EROWS // NS, 128), jnp.float32),
            pltpu.VMEM((TPB,), jnp.float32),
            pltpu.SemaphoreType.DMA,
        ],
    )
    return run(widx2d)


def _sc_agg(idxw, w2d, hw_flat):
    H = 128
    half = EROWS // (NC * NS)
    NG = half // 8

    def body(idxw_hbm, w_hbm, hw_hbm, out_hbm,
             acc_sh, idxA, idxB, wA, wB, rows0, rows1,
             gsem0, gsem1, ssem0, ssem1, isemA, isemB):
        cid = lax.axis_index("c")
        sid = lax.axis_index("s")
        rows = (rows0, rows1)
        gsem = (gsem0, gsem1)
        ssem = (ssem0, ssem1)

        @pl.loop(0, 128)
        def _z(i):
            for j in range(H // L):
                rows0[i, pl.ds(j * L, L)] = jnp.zeros((L,), jnp.float32)
        for k in range(N_ACC // NS // 128):
            pltpu.sync_copy(rows0,
                            acc_sh.at[pl.ds(sid * (N_ACC // NS) + k * 128, 128)])
        plsc.subcore_barrier()

        base = cid * (EROWS // NC) + sid * half

        def i_start(blk, wblk, sem, ch):
            pltpu.async_copy(idxw_hbm.at[pl.ds(base + ch, 4)], blk, sem)
            pltpu.async_copy(w_hbm.at[pl.ds(base + ch, 4)], wblk, sem)

        def i_wait(blk, wblk, sem, ch):
            pltpu.make_async_copy(idxw_hbm.at[pl.ds(base + ch, 4)], blk,
                                  sem).wait()
            pltpu.make_async_copy(w_hbm.at[pl.ds(base + ch, 4)], wblk,
                                  sem).wait()

        def g_start(blk, r, b):
            pltpu.async_copy(hw_hbm.at[blk.at[r, 0]], rows[b], gsem[b])

        def g_wait(blk, r, b):
            pltpu.make_async_copy(hw_hbm.at[blk.at[r, 0]], rows[b],
                                  gsem[b]).wait()

        def s_start(blk, r, b):
            pltpu.async_copy(rows[b], acc_sh.at[blk.at[r, 1]], ssem[b],
                             add=True)

        def s_wait(blk, r, b):
            pltpu.make_async_copy(rows[b], acc_sh.at[blk.at[r, 1]],
                                  ssem[b]).wait()

        def compute(blk, wblk, r, b):
            @pl.loop(0, 128 // L)
            def _grp(g):
                wv = wblk[r, pl.ds(g * L, L)]
                for e in range(L):
                    w = wv[e]
                    for j in range(H // L):
                        sl = pl.ds(j * L, L)
                        rows[b][g * L + e, sl] = rows[b][g * L + e, sl] * w

        i_start(idxA, wA, isemA, 0)
        i_start(idxB, wB, isemB, 4)
        i_wait(idxA, wA, isemA, 0)
        g_start(idxA, 0, 0)

        @pl.loop(0, NG)
        def _grp8(p):
            ch0 = p * 8
            for k in range(8):
                b = k % 2
                blk, wblk = (idxA, wA) if k < 4 else (idxB, wB)
                r = k % 4
                g_wait(blk, r, b)
                if k > 0:
                    blkp = idxA if (k - 1) < 4 else idxB
                    s_wait(blkp, (k - 1) % 4, 1 - b)
                else:
                    @pl.when(p > 0)
                    def _():
                        s_wait(idxB, 3, 1)
                        i_start(idxB, wB, isemB, ch0 + 4)
                if k == 4:
                    @pl.when(p < NG - 1)
                    def _():
                        i_start(idxA, wA, isemA, ch0 + 8)
                if k == 3:
                    i_wait(idxB, wB, isemB, ch0 + 4)
                if k < 7:
                    nblk = idxA if (k + 1) < 4 else idxB
                    g_start(nblk, (k + 1) % 4, 1 - b)
                else:
                    @pl.when(p < NG - 1)
                    def _():
                        i_wait(idxA, wA, isemA, ch0 + 8)
                        g_start(idxA, 0, 1 - b)
                compute(blk, wblk, r, b)
                s_start(blk, r, b)

        s_wait(idxB, 3, 1)
        plsc.subcore_barrier()

        for k in range(N_ACC // NS // 128):
            off = sid * (N_ACC // NS) + k * 128
            pltpu.sync_copy(acc_sh.at[pl.ds(off, 128)],
                            out_hbm.at[cid, pl.ds(off, 128)])

    run = pl.kernel(
        body,
        out_type=jax.ShapeDtypeStruct((NC, N_ACC, 128), jnp.float32),
        mesh=_mesh(),
        scratch_types=[
            pltpu.VMEM_SHARED((N_ACC, H), jnp.float32),
            pltpu.VMEM((4, 2, 128), jnp.int32),
            pltpu.VMEM((4, 2, 128), jnp.int32),
            pltpu.VMEM((4, 128), jnp.float32),
            pltpu.VMEM((4, 128), jnp.float32),
            pltpu.VMEM((128, H), jnp.float32),
            pltpu.VMEM((128, H), jnp.float32),
            pltpu.SemaphoreType.DMA,
            pltpu.SemaphoreType.DMA,
            pltpu.SemaphoreType.DMA,
            pltpu.SemaphoreType.DMA,
            pltpu.SemaphoreType.DMA,
            pltpu.SemaphoreType.DMA,
        ],
    )
    return run(idxw, w2d, hw_flat)


def _tc_mm1(x, wcat):
    def body(x_ref, w_ref, o_ref):
        o_ref[...] = jnp.dot(x_ref[...], w_ref[...],
                             preferred_element_type=jnp.float32)
    return pl.pallas_call(
        body,
        grid=(50,),
        in_specs=[
            pl.BlockSpec((200, 128), lambda i: (i, 0)),
            pl.BlockSpec((128, NINE * 128), lambda i: (0, 0)),
        ],
        out_specs=pl.BlockSpec((200, NINE * 128), lambda i: (i, 0)),
        out_shape=jax.ShapeDtypeStruct((N, NINE * 128), jnp.float32),
    )(x, wcat)


def _tc_fuse2(hw1, parts, b1, wcat2):
    def body(hw_ref, p_ref, b_ref, w_ref, o_ref):
        h = jax.nn.relu(hw_ref[...] + b_ref[0] + p_ref[0] + p_ref[1])
        o_ref[...] = jnp.dot(h, w_ref[...], preferred_element_type=jnp.float32)
    return pl.pallas_call(
        body,
        grid=(50,),
        in_specs=[
            pl.BlockSpec((200, 128), lambda i: (i, 0)),
            pl.BlockSpec((NC, 200, 128), lambda i: (0, i, 0)),
            pl.BlockSpec((1, 128), lambda i: (0, 0)),
            pl.BlockSpec((128, NINE * 128), lambda i: (0, 0)),
        ],
        out_specs=pl.BlockSpec((200, NINE * 128), lambda i: (i, 0)),
        out_shape=jax.ShapeDtypeStruct((N, NINE * 128), jnp.float32),
    )(hw1, parts, b1, wcat2)


def _tc_fuse3(hw2, parts, b2, ws, bs):
    def body(hw_ref, p_ref, b_ref, w_ref, bs_ref, o_ref):
        h = jax.nn.relu(hw_ref[:, :64] + b_ref[0]
                        + p_ref[0, :, :64] + p_ref[1, :, :64])
        raw = jnp.dot(h, w_ref[...], preferred_element_type=jnp.float32)
        o_ref[...] = jax.nn.sigmoid(raw + bs_ref[0])
    return pl.pallas_call(
        body,
        grid=(50,),
        in_specs=[
            pl.BlockSpec((200, 128), lambda i: (i, 0)),
            pl.BlockSpec((NC, 200, 128), lambda i: (0, i, 0)),
            pl.BlockSpec((1, 64), lambda i: (0, 0)),
            pl.BlockSpec((64, 1), lambda i: (0, 0)),
            pl.BlockSpec((1, 1), lambda i: (0, 0)),
        ],
        out_specs=pl.BlockSpec((200, 1), lambda i: (i, 0)),
        out_shape=jax.ShapeDtypeStruct((N, 1), jnp.float32),
    )(hw2, parts, b2, ws, bs)


def kernel(x, edge_index, edge_type, W1, root1, b1, W2, root2, b2, Ws, bs):
    src = edge_index[0].astype(jnp.int32)
    dst = edge_index[1].astype(jnp.int32)
    typ = edge_type.astype(jnp.int32)

    pad = E_PAD - E
    ar = jnp.arange(pad, dtype=jnp.int32)
    widx = jnp.concatenate([typ * N + dst, 80000 + ar % (CNT_TAB - 80000)])
    rowidx = jnp.concatenate([src * NINE + typ + 1, ar % (N * NINE)])
    dstp = jnp.concatenate([dst, N + ar % (N_ACC - N)])
    widx2d = widx.reshape(EROWS, 128)
    rowidx2d = rowidx.reshape(EROWS, 128)
    dst2d = dstp.reshape(EROWS, 128)

    wcat1 = jnp.concatenate(
        [root1[:, None, :], jnp.moveaxis(W1, 0, 1)], axis=1).reshape(128, NINE * 128)
    wcat2 = jnp.pad(
        jnp.concatenate([root2[:, None, :], jnp.moveaxis(W2, 0, 1)], axis=1),
        ((0, 0), (0, 0), (0, 64))).reshape(128, NINE * 128)

    hw1 = _tc_mm1(x, wcat1)
    w2d = _sc_weights(widx2d)
    idxw = jnp.stack([rowidx2d, dst2d], axis=1)
    parts1 = _sc_agg(idxw, w2d, hw1.reshape(N * NINE, 128))
    hw2 = _tc_fuse2(hw1, parts1, b1.reshape(1, 128), wcat2)
    parts2 = _sc_agg(idxw, w2d, hw2.reshape(N * NINE, 128))
    out = _tc_fuse3(hw2, parts2, b2.reshape(1, 64),
                    Ws, bs.reshape(1, 1))
    return jnp.squeeze(out, axis=-1)
